# baseline (device time: 32459 ns/iter reference)
import jax
import jax.numpy as jnp
from jax import lax
from jax.experimental import pallas as pl
from jax.experimental.pallas import tpu as pltpu

N_DEV = 4
N_LAYERS = 3
R = 4


def kernel(x, Win0, Wout0, Win1, Wout1, Win2, Wout2):
    b, d = x.shape
    ch = b // R

    def body(x_ref, win0_ref, wout0_ref, win1_ref, wout1_ref, win2_ref,
             wout2_ref, out_ref, comm_a, comm_b, p_ref, sa_ref,
             send_a, recv_a, send_b, recv_b):
        my_pos = lax.axis_index("i")
        partner_a = my_pos ^ 1
        partner_b = 3 - my_pos

        barrier_sem = pltpu.get_barrier_semaphore()
        for pid in (partner_a, partner_b):
            pl.semaphore_signal(
                barrier_sem, inc=1,
                device_id=(pid,), device_id_type=pl.DeviceIdType.MESH,
            )
        pl.semaphore_wait(barrier_sem, 2)

        wins = [win0_ref, win1_ref, win2_ref]
        wouts = [wout0_ref, wout1_ref, wout2_ref]
        rdma_a = {}
        rdma_b = {}

        def start_a(l, r):
            dsc = pltpu.make_async_remote_copy(
                src_ref=p_ref.at[l, r],
                dst_ref=comm_a.at[l, r],
                send_sem=send_a.at[l, r],
                recv_sem=recv_a.at[l, r],
                device_id=(partner_a,),
                device_id_type=pl.DeviceIdType.MESH,
            )
            dsc.start()
            rdma_a[(l, r)] = dsc

        def finish_a_start_b(l, r):
            rdma_a[(l, r)].wait_recv()
            sa_ref[l, r] = p_ref[l, r] + comm_a[l, r]
            dsc = pltpu.make_async_remote_copy(
                src_ref=sa_ref.at[l, r],
                dst_ref=comm_b.at[l, r],
                send_sem=send_b.at[l, r],
                recv_sem=recv_b.at[l, r],
                device_id=(partner_b,),
                device_id_type=pl.DeviceIdType.MESH,
            )
            dsc.start()
            rdma_b[(l, r)] = dsc

        for l in range(N_LAYERS):
            for r in range(R):
                if l == 0:
                    xc = x_ref[pl.ds(r * ch, ch), :]
                else:
                    rdma_b[(l - 1, r)].wait_recv()
                    xc = sa_ref[l - 1, r] + comm_b[l - 1, r]
                h = jnp.maximum(
                    jnp.dot(xc, wins[l][:, :],
                            preferred_element_type=jnp.float32),
                    0.0,
                )
                p_ref[l, r] = jnp.dot(
                    h, wouts[l][:, :], preferred_element_type=jnp.float32
                )
                start_a(l, r)
                if r > 0:
                    finish_a_start_b(l, r - 1)
            finish_a_start_b(l, R - 1)

        for r in range(R):
            rdma_b[(N_LAYERS - 1, r)].wait_recv()
            out_ref[pl.ds(r * ch, ch), :] = (
                sa_ref[N_LAYERS - 1, r] + comm_b[N_LAYERS - 1, r]
            )

        for dsc in list(rdma_a.values()) + list(rdma_b.values()):
            dsc.wait_send()

    return pl.pallas_call(
        body,
        out_shape=jax.ShapeDtypeStruct((b, d), jnp.float32),
        in_specs=[pl.BlockSpec(memory_space=pltpu.VMEM)] * 7,
        out_specs=pl.BlockSpec(memory_space=pltpu.VMEM),
        scratch_shapes=[
            pltpu.VMEM((N_LAYERS, R, ch, d), jnp.float32),
            pltpu.VMEM((N_LAYERS, R, ch, d), jnp.float32),
            pltpu.VMEM((N_LAYERS, R, ch, d), jnp.float32),
            pltpu.VMEM((N_LAYERS, R, ch, d), jnp.float32),
            pltpu.SemaphoreType.DMA((N_LAYERS, R)),
            pltpu.SemaphoreType.DMA((N_LAYERS, R)),
            pltpu.SemaphoreType.DMA((N_LAYERS, R)),
            pltpu.SemaphoreType.DMA((N_LAYERS, R)),
        ],
        compiler_params=pltpu.CompilerParams(collective_id=0),
    )(x, Win0, Wout0, Win1, Wout1, Win2, Wout2)
